# Initial kernel scaffold; baseline (speedup 1.0000x reference)
import jax
import jax.numpy as jnp
from jax.experimental import pallas as pl


def kernel(x_project, x_company, edge_index_p2c, edge_index_c2p, W_project, b_project, W_company, b_company):
    raise NotImplementedError("write your pallas kernel here")


# speedup vs baseline: 6.2863x; 6.2863x over previous
import functools

import jax
import jax.numpy as jnp
from jax import lax
from jax.experimental import pallas as pl
from jax.experimental.pallas import tpu as pltpu
from jax.experimental.pallas import tpu_sc as plsc

N = 10000
D = 128
E = 320000
NS = 16
HALF = 5120
DUMP = 128
AHALF = HALF + DUMP
RPT = AHALF // NS
EPT = E // NS
CHUNK = 80
NCHUNK = EPT // CHUNK

_mesh = plsc.VectorSubcoreMesh(core_axis_name="c", subcore_axis_name="s")

_ACC_OUT = jax.ShapeDtypeStruct((2, AHALF, D), jnp.float32)


def _redirect(cidx, lo):
    for j in range(CHUNK // 16):
        v = cidx[pl.ds(j * 16, 16)]
        local = v - lo
        oob = (local < 0) | (local >= HALF)
        dump = HALF + (v & (DUMP - 1))
        cidx[pl.ds(j * 16, 16)] = jnp.where(oob, dump, local)


def _zero_acc(zeros_hbm, stage_v, acc, sid):
    pltpu.sync_copy(zeros_hbm, stage_v)
    pltpu.sync_copy(stage_v, acc.at[pl.ds(sid * RPT, RPT)])


def _copy_out(acc, stage_v, out_hbm, h, sid):
    pltpu.sync_copy(acc.at[pl.ds(sid * RPT, RPT)], stage_v)
    pltpu.sync_copy(stage_v, out_hbm.at[h, pl.ds(sid * RPT, RPT)])


@functools.partial(
    pl.kernel,
    mesh=_mesh,
    out_type=[_ACC_OUT, _ACC_OUT],
    scratch_types=[
        pltpu.VMEM((CHUNK,), jnp.int32),
        pltpu.VMEM((CHUNK, D), jnp.float32),
        pltpu.VMEM((RPT, D), jnp.float32),
        pltpu.VMEM_SHARED((AHALF, D), jnp.float32),
    ],
)
def _sc_degree(idx_a, idx_b, ones_hbm, zeros_hbm, out_a, out_b,
               cidx, ones_v, stage_v, acc):
    cid = lax.axis_index("c")
    sid = lax.axis_index("s")
    base = sid * EPT

    def run(idx_hbm, out_hbm):
        pltpu.sync_copy(ones_hbm, ones_v)
        for h in range(2):
            lo = h * HALF
            _zero_acc(zeros_hbm, stage_v, acc, sid)
            plsc.subcore_barrier()

            def body(i, _):
                pltpu.sync_copy(idx_hbm.at[pl.ds(base + i * CHUNK, CHUNK)], cidx)
                _redirect(cidx, lo)
                pltpu.sync_copy(ones_v, acc.at[cidx], add=True)
                return 0

            lax.fori_loop(0, NCHUNK, body, 0)
            plsc.subcore_barrier()
            _copy_out(acc, stage_v, out_hbm, h, sid)

    @pl.when(cid == 0)
    def _():
        run(idx_a, out_a)

    @pl.when(cid == 1)
    def _():
        run(idx_b, out_b)


@functools.partial(
    pl.kernel,
    mesh=_mesh,
    out_type=[_ACC_OUT, _ACC_OUT],
    scratch_types=[
        pltpu.VMEM((CHUNK,), jnp.int32),
        pltpu.VMEM((CHUNK,), jnp.int32),
        pltpu.VMEM((CHUNK,), jnp.int32),
        pltpu.VMEM((CHUNK,), jnp.int32),
        pltpu.VMEM((CHUNK, D), jnp.float32),
        pltpu.VMEM((CHUNK, D), jnp.float32),
        pltpu.VMEM((RPT, D), jnp.float32),
        pltpu.VMEM_SHARED((AHALF, D), jnp.float32),
        pltpu.SemaphoreType.DMA,
        pltpu.SemaphoreType.DMA,
    ],
)
def _sc_propagate(tab_p, tab_c, row1, col1, row2, col2, zeros_hbm,
                  out_c, out_p,
                  ridx0, cidx0, ridx1, cidx1, rows0, rows1,
                  stage_v, acc, sem0, sem1):
    cid = lax.axis_index("c")
    sid = lax.axis_index("s")
    base = sid * EPT
    bufs = ((ridx0, cidx0, rows0, sem0), (ridx1, cidx1, rows1, sem1))

    def run(tab_hbm, row_hbm, col_hbm, out_hbm):
        def fetch(i, lo, ridx, cidx, rows, sem):
            off = base + i * CHUNK
            pltpu.sync_copy(row_hbm.at[pl.ds(off, CHUNK)], ridx)
            pltpu.sync_copy(col_hbm.at[pl.ds(off, CHUNK)], cidx)
            _redirect(cidx, lo)
            pltpu.async_copy(tab_hbm.at[ridx], rows, sem)

        for h in range(2):
            lo = h * HALF
            _zero_acc(zeros_hbm, stage_v, acc, sid)
            plsc.subcore_barrier()
            for b, (ridx, cidx, rows, sem) in enumerate(bufs):
                fetch(b, lo, ridx, cidx, rows, sem)

            def body(r, _):
                for b, (ridx, cidx, rows, sem) in enumerate(bufs):
                    i = 2 * r + b
                    pltpu.make_async_copy(tab_hbm.at[ridx], rows, sem).wait()
                    pltpu.sync_copy(rows, acc.at[cidx], add=True)

                    @pl.when(i + 2 < NCHUNK)
                    def _():
                        fetch(i + 2, lo, ridx, cidx, rows, sem)

                return 0

            lax.fori_loop(0, NCHUNK // 2, body, 0)
            plsc.subcore_barrier()
            _copy_out(acc, stage_v, out_hbm, h, sid)

    @pl.when(cid == 0)
    def _():
        run(tab_p, row1, col1, out_c)

    @pl.when(cid == 1)
    def _():
        run(tab_c, row2, col2, out_p)


_B = 1000


def _inv_sqrt(deg):
    return jnp.where(deg > 0.0, lax.rsqrt(jnp.maximum(deg, 1e-30)), 0.0)


def _node_specs(n):
    return [pl.BlockSpec((_B, D), lambda i: (i, 0)) for _ in range(n)]


def _tc_project_body(xp_ref, wp_ref, bp_ref, xc_ref, wc_ref, bc_ref,
                     dsp_ref, dsc_ref,
                     hp_ref, hc_ref, ap_ref, ac_ref):
    hp = lax.dot_general(xp_ref[...], wp_ref[...],
                         (((1,), (1,)), ((), ())),
                         preferred_element_type=jnp.float32) + bp_ref[...]
    hc = lax.dot_general(xc_ref[...], wc_ref[...],
                         (((1,), (1,)), ((), ())),
                         preferred_element_type=jnp.float32) + bc_ref[...]
    hp_ref[...] = hp
    hc_ref[...] = hc
    ap_ref[...] = hp * _inv_sqrt(dsp_ref[...])
    ac_ref[...] = hc * _inv_sqrt(dsc_ref[...])


def _tc_project(x_p, W_p, b_p, x_c, W_c, b_c, dsp, dsc):
    w_spec = pl.BlockSpec((D, D), lambda i: (0, 0))
    b_spec = pl.BlockSpec((1, D), lambda i: (0, 0))
    out = jax.ShapeDtypeStruct((N, D), jnp.float32)
    return pl.pallas_call(
        _tc_project_body,
        grid=(N // _B,),
        in_specs=(_node_specs(1) + [w_spec, b_spec] + _node_specs(1)
                  + [w_spec, b_spec] + _node_specs(2)),
        out_specs=_node_specs(4),
        out_shape=[out, out, out, out],
    )(x_p, W_p, b_p, x_c, W_c, b_c, dsp, dsc)


def _tc_rescale_body(accc_ref, accp_ref, ddc_ref, ddp_ref, dsc_ref, dsp_ref,
                     hc_ref, hp_ref, ac_ref, ap_ref):
    hc = accc_ref[...] * _inv_sqrt(ddc_ref[...])
    hp = accp_ref[...] * _inv_sqrt(ddp_ref[...])
    hc_ref[...] = hc
    hp_ref[...] = hp
    ac_ref[...] = hc * _inv_sqrt(dsc_ref[...])
    ap_ref[...] = hp * _inv_sqrt(dsp_ref[...])


def _tc_rescale(acc_c, acc_p, ddc, ddp, dsc, dsp):
    out = jax.ShapeDtypeStruct((N, D), jnp.float32)
    return pl.pallas_call(
        _tc_rescale_body,
        grid=(N // _B,),
        in_specs=_node_specs(6),
        out_specs=_node_specs(4),
        out_shape=[out, out, out, out],
    )(acc_c, acc_p, ddc, ddp, dsc, dsp)


def _tc_finalize_body(hp0_ref, hp1_ref, accp_ref, hc0_ref, hc1_ref, accc_ref,
                      ddp_ref, ddc_ref, op_ref, oc_ref):
    hp2 = accp_ref[...] * _inv_sqrt(ddp_ref[...])
    hc2 = accc_ref[...] * _inv_sqrt(ddc_ref[...])
    mp = (hp0_ref[...] + hp1_ref[...] + hp2) * (1.0 / 3.0)
    mc = (hc0_ref[...] + hc1_ref[...] + hc2) * (1.0 / 3.0)
    np_ = jnp.sqrt(jnp.sum(mp * mp, axis=1, keepdims=True))
    nc_ = jnp.sqrt(jnp.sum(mc * mc, axis=1, keepdims=True))
    op_ref[...] = mp / jnp.maximum(np_, 1e-12)
    oc_ref[...] = mc / jnp.maximum(nc_, 1e-12)


def _tc_finalize(hp0, hp1, acc_p2, hc0, hc1, acc_c2, ddp, ddc):
    out = jax.ShapeDtypeStruct((N, D), jnp.float32)
    return pl.pallas_call(
        _tc_finalize_body,
        grid=(N // _B,),
        in_specs=_node_specs(8),
        out_specs=_node_specs(2),
        out_shape=[out, out],
    )(hp0, hp1, acc_p2, hc0, hc1, acc_c2, ddp, ddc)


def _unhalf(o):
    return jnp.concatenate([o[0, :HALF], o[1, :HALF]], axis=0)[:N]


def kernel(x_project, x_company, edge_index_p2c, edge_index_c2p,
           W_project, b_project, W_company, b_company):
    row1 = edge_index_p2c[0]
    col1 = edge_index_p2c[1]
    row2 = edge_index_c2p[0]
    col2 = edge_index_c2p[1]

    ones_d = jnp.ones((CHUNK, D), jnp.float32)
    zeros_d = jnp.zeros((RPT, D), jnp.float32)
    b_p2 = b_project.reshape(1, D)
    b_c2 = b_company.reshape(1, D)

    ddc_r, ddp_r = _sc_degree(col1, col2, ones_d, zeros_d)
    dsp_r, dsc_r = _sc_degree(row1, row2, ones_d, zeros_d)
    ddc, ddp = _unhalf(ddc_r), _unhalf(ddp_r)
    dsp, dsc = _unhalf(dsp_r), _unhalf(dsc_r)

    hp0, hc0, ap0, ac0 = _tc_project(x_project, W_project, b_p2,
                                     x_company, W_company, b_c2, dsp, dsc)

    accc1, accp1 = _sc_propagate(ap0, ac0, row1, col1, row2, col2, zeros_d)
    hc1, hp1, ac1, ap1 = _tc_rescale(_unhalf(accc1), _unhalf(accp1),
                                     ddc, ddp, dsc, dsp)

    accc2, accp2 = _sc_propagate(ap1, ac1, row1, col1, row2, col2, zeros_d)

    out_p, out_c = _tc_finalize(hp0, hp1, _unhalf(accp2),
                                hc0, hc1, _unhalf(accc2), ddp, ddc)
    return out_p, out_c


# speedup vs baseline: 7.6721x; 1.2204x over previous
import functools

import jax
import jax.numpy as jnp
from jax import lax
from jax.experimental import pallas as pl
from jax.experimental.pallas import tpu as pltpu
from jax.experimental.pallas import tpu_sc as plsc

N = 10000
D = 128
E = 320000
NS = 16
HALF = 5120
DUMP = 128
AHALF = HALF + DUMP
RPT = AHALF // NS
CHUNK = 128
NCHUNK = 158
EPT = NCHUNK * CHUNK
EPAD = EPT * NS
NPAD = 10240

_mesh = plsc.VectorSubcoreMesh(core_axis_name="c", subcore_axis_name="s")

_ACC_OUT = jax.ShapeDtypeStruct((2, AHALF, D), jnp.float32)


def _redirect(cidx, lo):
    for j in range(CHUNK // 16):
        v = cidx[pl.ds(j * 16, 16)]
        local = v - lo
        oob = (local < 0) | (local >= HALF)
        dump = HALF + (v & (DUMP - 1))
        cidx[pl.ds(j * 16, 16)] = jnp.where(oob, dump, local)


def _zero_acc(zeros_hbm, stage_v, acc, sid):
    pltpu.sync_copy(zeros_hbm, stage_v)
    pltpu.sync_copy(stage_v, acc.at[pl.ds(sid * RPT, RPT)])


def _copy_out(acc, stage_v, out_hbm, h, sid):
    pltpu.sync_copy(acc.at[pl.ds(sid * RPT, RPT)], stage_v)
    pltpu.sync_copy(stage_v, out_hbm.at[h, pl.ds(sid * RPT, RPT)])


@functools.partial(
    pl.kernel,
    mesh=_mesh,
    out_type=[_ACC_OUT, _ACC_OUT],
    scratch_types=[
        pltpu.VMEM((CHUNK,), jnp.int32),
        pltpu.VMEM((CHUNK, D), jnp.float32),
        pltpu.VMEM((RPT, D), jnp.float32),
        pltpu.VMEM_SHARED((AHALF, D), jnp.float32),
    ],
)
def _sc_degree(idx_a, idx_b, ones_hbm, zeros_hbm, out_a, out_b,
               cidx, ones_v, stage_v, acc):
    cid = lax.axis_index("c")
    sid = lax.axis_index("s")
    base = sid * EPT

    def run(idx_hbm, out_hbm):
        pltpu.sync_copy(ones_hbm, ones_v)
        for h in range(2):
            lo = h * HALF
            _zero_acc(zeros_hbm, stage_v, acc, sid)
            plsc.subcore_barrier()

            def body(i, _):
                pltpu.sync_copy(idx_hbm.at[pl.ds(base + i * CHUNK, CHUNK)], cidx)
                _redirect(cidx, lo)
                pltpu.sync_copy(ones_v, acc.at[cidx], add=True)
                return 0

            lax.fori_loop(0, NCHUNK, body, 0)
            plsc.subcore_barrier()
            _copy_out(acc, stage_v, out_hbm, h, sid)

    @pl.when(cid == 0)
    def _():
        run(idx_a, out_a)

    @pl.when(cid == 1)
    def _():
        run(idx_b, out_b)


@functools.partial(
    pl.kernel,
    mesh=_mesh,
    out_type=[_ACC_OUT, _ACC_OUT],
    scratch_types=[
        pltpu.VMEM((CHUNK,), jnp.int32),
        pltpu.VMEM((CHUNK,), jnp.int32),
        pltpu.VMEM((CHUNK,), jnp.int32),
        pltpu.VMEM((CHUNK,), jnp.int32),
        pltpu.VMEM((CHUNK, D), jnp.float32),
        pltpu.VMEM((CHUNK, D), jnp.float32),
        pltpu.VMEM((RPT, D), jnp.float32),
        pltpu.VMEM_SHARED((AHALF, D), jnp.float32),
        pltpu.SemaphoreType.DMA,
        pltpu.SemaphoreType.DMA,
    ],
)
def _sc_propagate(tab_p, tab_c, row1, col1, row2, col2, zeros_hbm,
                  out_c, out_p,
                  ridx0, cidx0, ridx1, cidx1, rows0, rows1,
                  stage_v, acc, sem0, sem1):
    cid = lax.axis_index("c")
    sid = lax.axis_index("s")
    base = sid * EPT
    bufs = ((ridx0, cidx0, rows0, sem0), (ridx1, cidx1, rows1, sem1))

    def run(tab_hbm, row_hbm, col_hbm, out_hbm):
        def fetch(i, lo, ridx, cidx, rows, sem):
            off = base + i * CHUNK
            pltpu.sync_copy(row_hbm.at[pl.ds(off, CHUNK)], ridx)
            pltpu.sync_copy(col_hbm.at[pl.ds(off, CHUNK)], cidx)
            _redirect(cidx, lo)
            pltpu.async_copy(tab_hbm.at[ridx], rows, sem)

        for h in range(2):
            lo = h * HALF
            _zero_acc(zeros_hbm, stage_v, acc, sid)
            plsc.subcore_barrier()
            for b, (ridx, cidx, rows, sem) in enumerate(bufs):
                fetch(b, lo, ridx, cidx, rows, sem)

            def body(r, _):
                for b, (ridx, cidx, rows, sem) in enumerate(bufs):
                    i = 2 * r + b
                    pltpu.make_async_copy(tab_hbm.at[ridx], rows, sem).wait()
                    pltpu.sync_copy(rows, acc.at[cidx], add=True)

                    @pl.when(i + 2 < NCHUNK)
                    def _():
                        fetch(i + 2, lo, ridx, cidx, rows, sem)

                return 0

            lax.fori_loop(0, NCHUNK // 2, body, 0)
            plsc.subcore_barrier()
            _copy_out(acc, stage_v, out_hbm, h, sid)

    @pl.when(cid == 0)
    def _():
        run(tab_p, row1, col1, out_c)

    @pl.when(cid == 1)
    def _():
        run(tab_c, row2, col2, out_p)


_B = 1000


def _inv_sqrt(deg):
    return jnp.where(deg > 0.0, lax.rsqrt(jnp.maximum(deg, 1e-30)), 0.0)


def _node_specs(n):
    return [pl.BlockSpec((_B, D), lambda i: (i, 0)) for _ in range(n)]


def _tc_project_body(xp_ref, wp_ref, bp_ref, xc_ref, wc_ref, bc_ref,
                     dsp_ref, dsc_ref,
                     hp_ref, hc_ref, ap_ref, ac_ref):
    hp = lax.dot_general(xp_ref[...], wp_ref[...],
                         (((1,), (1,)), ((), ())),
                         preferred_element_type=jnp.float32) + bp_ref[...]
    hc = lax.dot_general(xc_ref[...], wc_ref[...],
                         (((1,), (1,)), ((), ())),
                         preferred_element_type=jnp.float32) + bc_ref[...]
    hp_ref[...] = hp
    hc_ref[...] = hc
    ap_ref[...] = hp * _inv_sqrt(dsp_ref[...])
    ac_ref[...] = hc * _inv_sqrt(dsc_ref[...])


def _tc_project(x_p, W_p, b_p, x_c, W_c, b_c, dsp, dsc):
    w_spec = pl.BlockSpec((D, D), lambda i: (0, 0))
    b_spec = pl.BlockSpec((1, D), lambda i: (0, 0))
    out = jax.ShapeDtypeStruct((N, D), jnp.float32)
    return pl.pallas_call(
        _tc_project_body,
        grid=(N // _B,),
        in_specs=(_node_specs(1) + [w_spec, b_spec] + _node_specs(1)
                  + [w_spec, b_spec] + _node_specs(2)),
        out_specs=_node_specs(4),
        out_shape=[out, out, out, out],
    )(x_p, W_p, b_p, x_c, W_c, b_c, dsp, dsc)


def _tc_rescale_body(accc_ref, accp_ref, ddc_ref, ddp_ref, dsc_ref, dsp_ref,
                     hc_ref, hp_ref, ac_ref, ap_ref):
    hc = accc_ref[...] * _inv_sqrt(ddc_ref[...])
    hp = accp_ref[...] * _inv_sqrt(ddp_ref[...])
    hc_ref[...] = hc
    hp_ref[...] = hp
    ac_ref[...] = hc * _inv_sqrt(dsc_ref[...])
    ap_ref[...] = hp * _inv_sqrt(dsp_ref[...])


def _tc_rescale(acc_c, acc_p, ddc, ddp, dsc, dsp):
    out = jax.ShapeDtypeStruct((N, D), jnp.float32)
    return pl.pallas_call(
        _tc_rescale_body,
        grid=(N // _B,),
        in_specs=_node_specs(6),
        out_specs=_node_specs(4),
        out_shape=[out, out, out, out],
    )(acc_c, acc_p, ddc, ddp, dsc, dsp)


def _tc_finalize_body(hp0_ref, hp1_ref, accp_ref, hc0_ref, hc1_ref, accc_ref,
                      ddp_ref, ddc_ref, op_ref, oc_ref):
    hp2 = accp_ref[...] * _inv_sqrt(ddp_ref[...])
    hc2 = accc_ref[...] * _inv_sqrt(ddc_ref[...])
    mp = (hp0_ref[...] + hp1_ref[...] + hp2) * (1.0 / 3.0)
    mc = (hc0_ref[...] + hc1_ref[...] + hc2) * (1.0 / 3.0)
    np_ = jnp.sqrt(jnp.sum(mp * mp, axis=1, keepdims=True))
    nc_ = jnp.sqrt(jnp.sum(mc * mc, axis=1, keepdims=True))
    op_ref[...] = mp / jnp.maximum(np_, 1e-12)
    oc_ref[...] = mc / jnp.maximum(nc_, 1e-12)


def _tc_finalize(hp0, hp1, acc_p2, hc0, hc1, acc_c2, ddp, ddc):
    out = jax.ShapeDtypeStruct((N, D), jnp.float32)
    return pl.pallas_call(
        _tc_finalize_body,
        grid=(N // _B,),
        in_specs=_node_specs(8),
        out_specs=_node_specs(2),
        out_shape=[out, out],
    )(hp0, hp1, acc_p2, hc0, hc1, acc_c2, ddp, ddc)


def _unhalf(o):
    return jnp.concatenate([o[0, :HALF], o[1, :HALF]], axis=0)[:N]


def kernel(x_project, x_company, edge_index_p2c, edge_index_c2p,
           W_project, b_project, W_company, b_company):
    pad = 10000 + (jnp.arange(EPAD - E, dtype=jnp.int32) % (NPAD - N))

    def padded(a):
        return jnp.concatenate([a, pad])

    row1 = padded(edge_index_p2c[0])
    col1 = padded(edge_index_p2c[1])
    row2 = padded(edge_index_c2p[0])
    col2 = padded(edge_index_c2p[1])
    tab_pad = jnp.zeros((NPAD - N, D), jnp.float32)

    def padtab(t):
        return jnp.concatenate([t, tab_pad], axis=0)

    ones_d = jnp.ones((CHUNK, D), jnp.float32)
    zeros_d = jnp.zeros((RPT, D), jnp.float32)
    b_p2 = b_project.reshape(1, D)
    b_c2 = b_company.reshape(1, D)

    ddc_r, ddp_r = _sc_degree(col1, col2, ones_d, zeros_d)
    dsp_r, dsc_r = _sc_degree(row1, row2, ones_d, zeros_d)
    ddc, ddp = _unhalf(ddc_r), _unhalf(ddp_r)
    dsp, dsc = _unhalf(dsp_r), _unhalf(dsc_r)

    hp0, hc0, ap0, ac0 = _tc_project(x_project, W_project, b_p2,
                                     x_company, W_company, b_c2, dsp, dsc)

    accc1, accp1 = _sc_propagate(padtab(ap0), padtab(ac0),
                                 row1, col1, row2, col2, zeros_d)
    hc1, hp1, ac1, ap1 = _tc_rescale(_unhalf(accc1), _unhalf(accp1),
                                     ddc, ddp, dsc, dsp)

    accc2, accp2 = _sc_propagate(padtab(ap1), padtab(ac1),
                                 row1, col1, row2, col2, zeros_d)

    out_p, out_c = _tc_finalize(hp0, hp1, _unhalf(accp2),
                                hc0, hc1, _unhalf(accc2), ddp, ddc)
    return out_p, out_c


# speedup vs baseline: 10.9509x; 1.4274x over previous
import functools

import jax
import jax.numpy as jnp
from jax import lax
from jax.experimental import pallas as pl
from jax.experimental.pallas import tpu as pltpu
from jax.experimental.pallas import tpu_sc as plsc

N = 10000
D = 128
E = 320000
NS = 16
HALF = 5120
DUMP = 128
AHALF = HALF + DUMP
RPT = AHALF // NS
CHUNK = 128
NCHUNK = 160
EPT = NCHUNK * CHUNK
EPAD = EPT * NS
NPAD = 10240

_mesh = plsc.VectorSubcoreMesh(core_axis_name="c", subcore_axis_name="s")

_ACC_OUT = jax.ShapeDtypeStruct((2, AHALF, D), jnp.float32)


def _redirect(cidx, lo):
    for j in range(CHUNK // 16):
        v = cidx[pl.ds(j * 16, 16)]
        local = v - lo
        oob = (local < 0) | (local >= HALF)
        dump = HALF + (v & (DUMP - 1))
        cidx[pl.ds(j * 16, 16)] = jnp.where(oob, dump, local)


_PIECES = (128, 128, 72)


def _zero_acc(zeros_hbm, buf, acc, sid):
    pltpu.sync_copy(zeros_hbm, buf)
    o = 0
    for p in _PIECES:
        pltpu.sync_copy(buf.at[pl.ds(0, p)], acc.at[pl.ds(sid * RPT + o, p)])
        o += p


def _copy_out(acc, buf, out_hbm, h, sid):
    o = 0
    for p in _PIECES:
        pltpu.sync_copy(acc.at[pl.ds(sid * RPT + o, p)], buf.at[pl.ds(0, p)])
        pltpu.sync_copy(buf.at[pl.ds(0, p)], out_hbm.at[h, pl.ds(sid * RPT + o, p)])
        o += p


@functools.partial(
    pl.kernel,
    mesh=_mesh,
    out_type=[_ACC_OUT, _ACC_OUT],
    scratch_types=[
        pltpu.VMEM((CHUNK,), jnp.int32),
        pltpu.VMEM((CHUNK,), jnp.int32),
        pltpu.VMEM((CHUNK,), jnp.int32),
        pltpu.VMEM((CHUNK,), jnp.int32),
        pltpu.VMEM((CHUNK, D), jnp.float32),
        pltpu.VMEM((128, D), jnp.float32),
        pltpu.VMEM_SHARED((AHALF, D), jnp.float32),
        pltpu.SemaphoreType.DMA,
        pltpu.SemaphoreType.DMA,
        pltpu.SemaphoreType.DMA,
        pltpu.SemaphoreType.DMA,
    ],
)
def _sc_degree(idx_a, idx_b, ones_hbm, zeros_hbm, out_a, out_b,
               cidx0, cidx1, cidx2, cidx3, ones_v, stage_v, acc,
               ss0, ss1, ss2, ss3):
    cid = lax.axis_index("c")
    sid = lax.axis_index("s")
    base = sid * EPT
    bufs = ((cidx0, ss0), (cidx1, ss1), (cidx2, ss2), (cidx3, ss3))

    def run(idx_hbm, out_hbm):
        pltpu.sync_copy(ones_hbm, ones_v)

        def fetch(i, lo, b):
            cidx, _ = bufs[b]
            pltpu.sync_copy(idx_hbm.at[pl.ds(base + i * CHUNK, CHUNK)], cidx)
            _redirect(cidx, lo)

        def wait_scatter(b):
            cidx, ss = bufs[b]
            pltpu.make_async_copy(ones_v, acc.at[cidx], ss).wait()

        for h in range(2):
            lo = h * HALF
            _zero_acc(zeros_hbm, stage_v, acc, sid)
            plsc.subcore_barrier()
            fetch(0, lo, 0)
            fetch(1, lo, 1)

            def body(r, _):
                for k in range(4):
                    i = 4 * r + k
                    cidx, ss = bufs[k]

                    @pl.when(i >= 2)
                    def _():
                        wait_scatter((k - 2) % 4)

                    @pl.when(i + 2 < NCHUNK)
                    def _():
                        fetch(i + 2, lo, (k + 2) % 4)

                    pltpu.async_copy(ones_v, acc.at[cidx], ss, add=True)
                return 0

            lax.fori_loop(0, NCHUNK // 4, body, 0)
            wait_scatter(2)
            wait_scatter(3)
            plsc.subcore_barrier()
            _copy_out(acc, stage_v, out_hbm, h, sid)

    @pl.when(cid == 0)
    def _():
        run(idx_a, out_a)

    @pl.when(cid == 1)
    def _():
        run(idx_b, out_b)


@functools.partial(
    pl.kernel,
    mesh=_mesh,
    out_type=[_ACC_OUT, _ACC_OUT],
    scratch_types=[
        pltpu.VMEM((CHUNK,), jnp.int32),
        pltpu.VMEM((CHUNK,), jnp.int32),
        pltpu.VMEM((CHUNK,), jnp.int32),
        pltpu.VMEM((CHUNK,), jnp.int32),
        pltpu.VMEM((CHUNK,), jnp.int32),
        pltpu.VMEM((CHUNK,), jnp.int32),
        pltpu.VMEM((CHUNK,), jnp.int32),
        pltpu.VMEM((CHUNK,), jnp.int32),
        pltpu.VMEM((CHUNK, D), jnp.float32),
        pltpu.VMEM((CHUNK, D), jnp.float32),
        pltpu.VMEM((CHUNK, D), jnp.float32),
        pltpu.VMEM((CHUNK, D), jnp.float32),
        pltpu.VMEM_SHARED((AHALF, D), jnp.float32),
        pltpu.SemaphoreType.DMA,
        pltpu.SemaphoreType.DMA,
        pltpu.SemaphoreType.DMA,
        pltpu.SemaphoreType.DMA,
        pltpu.SemaphoreType.DMA,
        pltpu.SemaphoreType.DMA,
        pltpu.SemaphoreType.DMA,
        pltpu.SemaphoreType.DMA,
    ],
)
def _sc_propagate(tab_p, tab_c, row1, col1, row2, col2, zeros_hbm,
                  out_c, out_p,
                  ridx0, cidx0, ridx1, cidx1, ridx2, cidx2, ridx3, cidx3,
                  rows0, rows1, rows2, rows3, acc,
                  gs0, gs1, gs2, gs3, ss0, ss1, ss2, ss3):
    cid = lax.axis_index("c")
    sid = lax.axis_index("s")
    base = sid * EPT
    bufs = ((ridx0, cidx0, rows0, gs0, ss0), (ridx1, cidx1, rows1, gs1, ss1),
            (ridx2, cidx2, rows2, gs2, ss2), (ridx3, cidx3, rows3, gs3, ss3))

    def run(tab_hbm, row_hbm, col_hbm, out_hbm):
        def fetch(i, lo, b):
            ridx, cidx, rows, gs, _ = bufs[b]
            off = base + i * CHUNK
            pltpu.sync_copy(row_hbm.at[pl.ds(off, CHUNK)], ridx)
            pltpu.sync_copy(col_hbm.at[pl.ds(off, CHUNK)], cidx)
            _redirect(cidx, lo)
            pltpu.async_copy(tab_hbm.at[ridx], rows, gs)

        def wait_scatter(b):
            _, cidx, rows, _, ss = bufs[b]
            pltpu.make_async_copy(rows, acc.at[cidx], ss).wait()

        for h in range(2):
            lo = h * HALF
            _zero_acc(zeros_hbm, rows0, acc, sid)
            plsc.subcore_barrier()
            fetch(0, lo, 0)
            fetch(1, lo, 1)

            def body(r, _):
                for k in range(4):
                    i = 4 * r + k
                    ridx, cidx, rows, gs, ss = bufs[k]

                    @pl.when(i >= 2)
                    def _():
                        wait_scatter((k - 2) % 4)

                    @pl.when(i + 2 < NCHUNK)
                    def _():
                        fetch(i + 2, lo, (k + 2) % 4)

                    pltpu.make_async_copy(tab_hbm.at[ridx], rows, gs).wait()
                    pltpu.async_copy(rows, acc.at[cidx], ss, add=True)
                return 0

            lax.fori_loop(0, NCHUNK // 4, body, 0)
            wait_scatter(2)
            wait_scatter(3)
            plsc.subcore_barrier()
            _copy_out(acc, rows0, out_hbm, h, sid)

    @pl.when(cid == 0)
    def _():
        run(tab_p, row1, col1, out_c)

    @pl.when(cid == 1)
    def _():
        run(tab_c, row2, col2, out_p)


_B = 1000


def _inv_sqrt(deg):
    return jnp.where(deg > 0.0, lax.rsqrt(jnp.maximum(deg, 1e-30)), 0.0)


def _node_specs(n):
    return [pl.BlockSpec((_B, D), lambda i: (i, 0)) for _ in range(n)]


def _tc_project_body(xp_ref, wp_ref, bp_ref, xc_ref, wc_ref, bc_ref,
                     dsp_ref, dsc_ref,
                     hp_ref, hc_ref, ap_ref, ac_ref):
    hp = lax.dot_general(xp_ref[...], wp_ref[...],
                         (((1,), (1,)), ((), ())),
                         preferred_element_type=jnp.float32) + bp_ref[...]
    hc = lax.dot_general(xc_ref[...], wc_ref[...],
                         (((1,), (1,)), ((), ())),
                         preferred_element_type=jnp.float32) + bc_ref[...]
    hp_ref[...] = hp
    hc_ref[...] = hc
    ap_ref[...] = hp * _inv_sqrt(dsp_ref[...])
    ac_ref[...] = hc * _inv_sqrt(dsc_ref[...])


def _tc_project(x_p, W_p, b_p, x_c, W_c, b_c, dsp, dsc):
    w_spec = pl.BlockSpec((D, D), lambda i: (0, 0))
    b_spec = pl.BlockSpec((1, D), lambda i: (0, 0))
    out = jax.ShapeDtypeStruct((N, D), jnp.float32)
    return pl.pallas_call(
        _tc_project_body,
        grid=(N // _B,),
        in_specs=(_node_specs(1) + [w_spec, b_spec] + _node_specs(1)
                  + [w_spec, b_spec] + _node_specs(2)),
        out_specs=_node_specs(4),
        out_shape=[out, out, out, out],
    )(x_p, W_p, b_p, x_c, W_c, b_c, dsp, dsc)


def _tc_rescale_body(accc_ref, accp_ref, ddc_ref, ddp_ref, dsc_ref, dsp_ref,
                     hc_ref, hp_ref, ac_ref, ap_ref):
    hc = accc_ref[...] * _inv_sqrt(ddc_ref[...])
    hp = accp_ref[...] * _inv_sqrt(ddp_ref[...])
    hc_ref[...] = hc
    hp_ref[...] = hp
    ac_ref[...] = hc * _inv_sqrt(dsc_ref[...])
    ap_ref[...] = hp * _inv_sqrt(dsp_ref[...])


def _tc_rescale(acc_c, acc_p, ddc, ddp, dsc, dsp):
    out = jax.ShapeDtypeStruct((N, D), jnp.float32)
    return pl.pallas_call(
        _tc_rescale_body,
        grid=(N // _B,),
        in_specs=_node_specs(6),
        out_specs=_node_specs(4),
        out_shape=[out, out, out, out],
    )(acc_c, acc_p, ddc, ddp, dsc, dsp)


def _tc_finalize_body(hp0_ref, hp1_ref, accp_ref, hc0_ref, hc1_ref, accc_ref,
                      ddp_ref, ddc_ref, op_ref, oc_ref):
    hp2 = accp_ref[...] * _inv_sqrt(ddp_ref[...])
    hc2 = accc_ref[...] * _inv_sqrt(ddc_ref[...])
    mp = (hp0_ref[...] + hp1_ref[...] + hp2) * (1.0 / 3.0)
    mc = (hc0_ref[...] + hc1_ref[...] + hc2) * (1.0 / 3.0)
    np_ = jnp.sqrt(jnp.sum(mp * mp, axis=1, keepdims=True))
    nc_ = jnp.sqrt(jnp.sum(mc * mc, axis=1, keepdims=True))
    op_ref[...] = mp / jnp.maximum(np_, 1e-12)
    oc_ref[...] = mc / jnp.maximum(nc_, 1e-12)


def _tc_finalize(hp0, hp1, acc_p2, hc0, hc1, acc_c2, ddp, ddc):
    out = jax.ShapeDtypeStruct((N, D), jnp.float32)
    return pl.pallas_call(
        _tc_finalize_body,
        grid=(N // _B,),
        in_specs=_node_specs(8),
        out_specs=_node_specs(2),
        out_shape=[out, out],
    )(hp0, hp1, acc_p2, hc0, hc1, acc_c2, ddp, ddc)


def _unhalf(o):
    return jnp.concatenate([o[0, :HALF], o[1, :HALF]], axis=0)[:N]


def kernel(x_project, x_company, edge_index_p2c, edge_index_c2p,
           W_project, b_project, W_company, b_company):
    pad = 10000 + (jnp.arange(EPAD - E, dtype=jnp.int32) % (NPAD - N))

    def padded(a):
        return jnp.concatenate([a, pad])

    row1 = padded(edge_index_p2c[0])
    col1 = padded(edge_index_p2c[1])
    row2 = padded(edge_index_c2p[0])
    col2 = padded(edge_index_c2p[1])
    tab_pad = jnp.zeros((NPAD - N, D), jnp.float32)

    def padtab(t):
        return jnp.concatenate([t, tab_pad], axis=0)

    ones_d = jnp.ones((CHUNK, D), jnp.float32)
    zeros_d = jnp.zeros((128, D), jnp.float32)
    b_p2 = b_project.reshape(1, D)
    b_c2 = b_company.reshape(1, D)

    ddc_r, ddp_r = _sc_degree(col1, col2, ones_d, zeros_d)
    dsp_r, dsc_r = _sc_degree(row1, row2, ones_d, zeros_d)
    ddc, ddp = _unhalf(ddc_r), _unhalf(ddp_r)
    dsp, dsc = _unhalf(dsp_r), _unhalf(dsc_r)

    hp0, hc0, ap0, ac0 = _tc_project(x_project, W_project, b_p2,
                                     x_company, W_company, b_c2, dsp, dsc)

    accc1, accp1 = _sc_propagate(padtab(ap0), padtab(ac0),
                                 row1, col1, row2, col2, zeros_d)
    hc1, hp1, ac1, ap1 = _tc_rescale(_unhalf(accc1), _unhalf(accp1),
                                     ddc, ddp, dsc, dsp)

    accc2, accp2 = _sc_propagate(padtab(ap1), padtab(ac1),
                                 row1, col1, row2, col2, zeros_d)

    out_p, out_c = _tc_finalize(hp0, hp1, _unhalf(accp2),
                                hc0, hc1, _unhalf(accc2), ddp, ddc)
    return out_p, out_c


# speedup vs baseline: 18.5106x; 1.6903x over previous
import functools

import jax
import jax.numpy as jnp
from jax import lax
from jax.experimental import pallas as pl
from jax.experimental.pallas import tpu as pltpu
from jax.experimental.pallas import tpu_sc as plsc

N = 10000
D = 128
E = 320000
NS = 16
NPAD = 10240
RPT = NPAD // NS
CHUNK = 120
NCHUNK = 168
EPT = NCHUNK * CHUNK
EPAD = EPT * NS

_mesh = plsc.VectorSubcoreMesh(core_axis_name="c", subcore_axis_name="s")

_ACC_OUT = jax.ShapeDtypeStruct((NPAD, D), jnp.float32)


_PIECES = (120, 120, 120, 120, 120, 40)


def _zero_acc(zeros_hbm, buf, acc, sid):
    pltpu.sync_copy(zeros_hbm, buf)
    o = 0
    for p in _PIECES:
        pltpu.sync_copy(buf.at[pl.ds(0, p)], acc.at[pl.ds(sid * RPT + o, p)])
        o += p


def _copy_out(acc, buf, out_hbm, sid):
    o = 0
    for p in _PIECES:
        pltpu.sync_copy(acc.at[pl.ds(sid * RPT + o, p)], buf.at[pl.ds(0, p)])
        pltpu.sync_copy(buf.at[pl.ds(0, p)], out_hbm.at[pl.ds(sid * RPT + o, p)])
        o += p


@functools.partial(
    pl.kernel,
    mesh=_mesh,
    out_type=[_ACC_OUT, _ACC_OUT, _ACC_OUT, _ACC_OUT],
    scratch_types=[
        pltpu.VMEM((CHUNK,), jnp.int32),
        pltpu.VMEM((CHUNK,), jnp.int32),
        pltpu.VMEM((CHUNK,), jnp.int32),
        pltpu.VMEM((CHUNK, D), jnp.float32),
        pltpu.VMEM((CHUNK, D), jnp.float32),
        pltpu.VMEM_SHARED((NPAD, D), jnp.float32),
        pltpu.SemaphoreType.DMA,
        pltpu.SemaphoreType.DMA,
        pltpu.SemaphoreType.DMA,
    ],
)
def _sc_degree(col_a, row_a, col_b, row_b, ones_hbm, zeros_hbm,
               deg_ca, deg_ra, deg_cb, deg_rb,
               cidx0, cidx1, cidx2, ones_v, stage_v, acc, ss0, ss1, ss2):
    cid = lax.axis_index("c")
    sid = lax.axis_index("s")
    base = sid * EPT
    bufs = ((cidx0, ss0), (cidx1, ss1), (cidx2, ss2))

    def run(idx1_hbm, out1_hbm, idx2_hbm, out2_hbm):
        pltpu.sync_copy(ones_hbm, ones_v)
        for idx_hbm, out_hbm in ((idx1_hbm, out1_hbm), (idx2_hbm, out2_hbm)):
            def fetch(i, b):
                pltpu.sync_copy(idx_hbm.at[pl.ds(base + i * CHUNK, CHUNK)],
                                bufs[b][0])

            def wait_scatter(b):
                cidx, ss = bufs[b]
                pltpu.make_async_copy(ones_v, acc.at[cidx], ss).wait()

            _zero_acc(zeros_hbm, stage_v, acc, sid)
            plsc.subcore_barrier()
            fetch(0, 0)

            def body(r, _):
                for k in range(3):
                    i = 3 * r + k
                    cidx, ss = bufs[k]

                    @pl.when(i >= 2)
                    def _():
                        wait_scatter((k + 1) % 3)

                    @pl.when(i + 1 < NCHUNK)
                    def _():
                        fetch(i + 1, (k + 1) % 3)

                    pltpu.async_copy(ones_v, acc.at[cidx], ss, add=True)
                return 0

            lax.fori_loop(0, NCHUNK // 3, body, 0)
            wait_scatter((NCHUNK - 2) % 3)
            wait_scatter((NCHUNK - 1) % 3)
            plsc.subcore_barrier()
            _copy_out(acc, stage_v, out_hbm, sid)

    @pl.when(cid == 0)
    def _():
        run(col_a, deg_ca, row_a, deg_ra)

    @pl.when(cid == 1)
    def _():
        run(col_b, deg_cb, row_b, deg_rb)


@functools.partial(
    pl.kernel,
    mesh=_mesh,
    out_type=[_ACC_OUT, _ACC_OUT],
    scratch_types=[
        pltpu.VMEM((CHUNK,), jnp.int32),
        pltpu.VMEM((CHUNK,), jnp.int32),
        pltpu.VMEM((CHUNK,), jnp.int32),
        pltpu.VMEM((CHUNK,), jnp.int32),
        pltpu.VMEM((CHUNK,), jnp.int32),
        pltpu.VMEM((CHUNK,), jnp.int32),
        pltpu.VMEM((CHUNK, D), jnp.float32),
        pltpu.VMEM((CHUNK, D), jnp.float32),
        pltpu.VMEM((CHUNK, D), jnp.float32),
        pltpu.VMEM_SHARED((NPAD, D), jnp.float32),
        pltpu.SemaphoreType.DMA,
        pltpu.SemaphoreType.DMA,
        pltpu.SemaphoreType.DMA,
        pltpu.SemaphoreType.DMA,
        pltpu.SemaphoreType.DMA,
        pltpu.SemaphoreType.DMA,
    ],
)
def _sc_propagate(tab_p, tab_c, row1, col1, row2, col2, zeros_hbm,
                  out_c, out_p,
                  ridx0, cidx0, ridx1, cidx1, ridx2, cidx2,
                  rows0, rows1, rows2, acc,
                  gs0, gs1, gs2, ss0, ss1, ss2):
    cid = lax.axis_index("c")
    sid = lax.axis_index("s")
    base = sid * EPT
    bufs = ((ridx0, cidx0, rows0, gs0, ss0),
            (ridx1, cidx1, rows1, gs1, ss1),
            (ridx2, cidx2, rows2, gs2, ss2))

    def run(tab_hbm, row_hbm, col_hbm, out_hbm):
        def fetch(i, b):
            ridx, cidx, rows, gs, _ = bufs[b]
            off = base + i * CHUNK
            pltpu.sync_copy(row_hbm.at[pl.ds(off, CHUNK)], ridx)
            pltpu.sync_copy(col_hbm.at[pl.ds(off, CHUNK)], cidx)
            pltpu.async_copy(tab_hbm.at[ridx], rows, gs)

        def wait_scatter(b):
            _, cidx, rows, _, ss = bufs[b]
            pltpu.make_async_copy(rows, acc.at[cidx], ss).wait()

        _zero_acc(zeros_hbm, rows0, acc, sid)
        plsc.subcore_barrier()
        fetch(0, 0)

        def body(r, _):
            for k in range(3):
                i = 3 * r + k
                ridx, cidx, rows, gs, ss = bufs[k]

                @pl.when(i >= 2)
                def _():
                    wait_scatter((k + 1) % 3)

                @pl.when(i + 1 < NCHUNK)
                def _():
                    fetch(i + 1, (k + 1) % 3)

                pltpu.make_async_copy(tab_hbm.at[ridx], rows, gs).wait()
                pltpu.async_copy(rows, acc.at[cidx], ss, add=True)
            return 0

        lax.fori_loop(0, NCHUNK // 3, body, 0)
        wait_scatter((NCHUNK - 2) % 3)
        wait_scatter((NCHUNK - 1) % 3)
        plsc.subcore_barrier()
        _copy_out(acc, rows1, out_hbm, sid)

    @pl.when(cid == 0)
    def _():
        run(tab_p, row1, col1, out_c)

    @pl.when(cid == 1)
    def _():
        run(tab_c, row2, col2, out_p)


_B = 1000


def _inv_sqrt(deg):
    return jnp.where(deg > 0.0, lax.rsqrt(jnp.maximum(deg, 1e-30)), 0.0)


def _node_specs(n):
    return [pl.BlockSpec((_B, D), lambda i: (i, 0)) for _ in range(n)]


def _tc_project_body(xp_ref, wp_ref, bp_ref, xc_ref, wc_ref, bc_ref,
                     dsp_ref, dsc_ref,
                     hp_ref, hc_ref, ap_ref, ac_ref):
    hp = lax.dot_general(xp_ref[...], wp_ref[...],
                         (((1,), (1,)), ((), ())),
                         preferred_element_type=jnp.float32) + bp_ref[...]
    hc = lax.dot_general(xc_ref[...], wc_ref[...],
                         (((1,), (1,)), ((), ())),
                         preferred_element_type=jnp.float32) + bc_ref[...]
    hp_ref[...] = hp
    hc_ref[...] = hc
    ap_ref[...] = hp * _inv_sqrt(dsp_ref[...])
    ac_ref[...] = hc * _inv_sqrt(dsc_ref[...])


def _tc_project(x_p, W_p, b_p, x_c, W_c, b_c, dsp, dsc):
    w_spec = pl.BlockSpec((D, D), lambda i: (0, 0))
    b_spec = pl.BlockSpec((1, D), lambda i: (0, 0))
    out = jax.ShapeDtypeStruct((N, D), jnp.float32)
    return pl.pallas_call(
        _tc_project_body,
        grid=(N // _B,),
        in_specs=(_node_specs(1) + [w_spec, b_spec] + _node_specs(1)
                  + [w_spec, b_spec] + _node_specs(2)),
        out_specs=_node_specs(4),
        out_shape=[out, out, out, out],
    )(x_p, W_p, b_p, x_c, W_c, b_c, dsp, dsc)


def _tc_rescale_body(accc_ref, accp_ref, ddc_ref, ddp_ref, dsc_ref, dsp_ref,
                     hc_ref, hp_ref, ac_ref, ap_ref):
    hc = accc_ref[...] * _inv_sqrt(ddc_ref[...])
    hp = accp_ref[...] * _inv_sqrt(ddp_ref[...])
    hc_ref[...] = hc
    hp_ref[...] = hp
    ac_ref[...] = hc * _inv_sqrt(dsc_ref[...])
    ap_ref[...] = hp * _inv_sqrt(dsp_ref[...])


def _tc_rescale(acc_c, acc_p, ddc, ddp, dsc, dsp):
    out = jax.ShapeDtypeStruct((N, D), jnp.float32)
    return pl.pallas_call(
        _tc_rescale_body,
        grid=(N // _B,),
        in_specs=_node_specs(6),
        out_specs=_node_specs(4),
        out_shape=[out, out, out, out],
    )(acc_c, acc_p, ddc, ddp, dsc, dsp)


def _tc_finalize_body(hp0_ref, hp1_ref, accp_ref, hc0_ref, hc1_ref, accc_ref,
                      ddp_ref, ddc_ref, op_ref, oc_ref):
    hp2 = accp_ref[...] * _inv_sqrt(ddp_ref[...])
    hc2 = accc_ref[...] * _inv_sqrt(ddc_ref[...])
    mp = (hp0_ref[...] + hp1_ref[...] + hp2) * (1.0 / 3.0)
    mc = (hc0_ref[...] + hc1_ref[...] + hc2) * (1.0 / 3.0)
    np_ = jnp.sqrt(jnp.sum(mp * mp, axis=1, keepdims=True))
    nc_ = jnp.sqrt(jnp.sum(mc * mc, axis=1, keepdims=True))
    op_ref[...] = mp / jnp.maximum(np_, 1e-12)
    oc_ref[...] = mc / jnp.maximum(nc_, 1e-12)


def _tc_finalize(hp0, hp1, acc_p2, hc0, hc1, acc_c2, ddp, ddc):
    out = jax.ShapeDtypeStruct((N, D), jnp.float32)
    return pl.pallas_call(
        _tc_finalize_body,
        grid=(N // _B,),
        in_specs=_node_specs(8),
        out_specs=_node_specs(2),
        out_shape=[out, out],
    )(hp0, hp1, acc_p2, hc0, hc1, acc_c2, ddp, ddc)


def kernel(x_project, x_company, edge_index_p2c, edge_index_c2p,
           W_project, b_project, W_company, b_company):
    pad = 10000 + (jnp.arange(EPAD - E, dtype=jnp.int32) % (NPAD - N))

    def padded(a):
        return jnp.concatenate([a, pad])

    row1 = padded(edge_index_p2c[0])
    col1 = padded(edge_index_p2c[1])
    row2 = padded(edge_index_c2p[0])
    col2 = padded(edge_index_c2p[1])
    tab_pad = jnp.zeros((NPAD - N, D), jnp.float32)

    def padtab(t):
        return jnp.concatenate([t, tab_pad], axis=0)

    ones_d = jnp.ones((CHUNK, D), jnp.float32)
    zeros_d = jnp.zeros((CHUNK, D), jnp.float32)
    b_p2 = b_project.reshape(1, D)
    b_c2 = b_company.reshape(1, D)

    ddc_r, dsp_r, ddp_r, dsc_r = _sc_degree(col1, row1, col2, row2,
                                            ones_d, zeros_d)
    ddc, ddp = ddc_r[:N], ddp_r[:N]
    dsp, dsc = dsp_r[:N], dsc_r[:N]

    hp0, hc0, ap0, ac0 = _tc_project(x_project, W_project, b_p2,
                                     x_company, W_company, b_c2, dsp, dsc)

    accc1, accp1 = _sc_propagate(padtab(ap0), padtab(ac0),
                                 row1, col1, row2, col2, zeros_d)
    hc1, hp1, ac1, ap1 = _tc_rescale(accc1[:N], accp1[:N],
                                     ddc, ddp, dsc, dsp)

    accc2, accp2 = _sc_propagate(padtab(ap1), padtab(ac1),
                                 row1, col1, row2, col2, zeros_d)

    out_p, out_c = _tc_finalize(hp0, hp1, accp2[:N],
                                hc0, hc1, accc2[:N], ddp, ddc)
    return out_p, out_c
